# baseline (speedup 1.0000x reference)
import functools

import jax
import jax.numpy as jnp
from jax import lax
from jax.experimental import pallas as pl
from jax.experimental.pallas import tpu as pltpu
from jax.experimental.pallas import tpu_sc as plsc

NC, NS, L = 2, 16, 16
NW = NC * NS
LAG = 16


def kernel(idx, emb_weight):
    B = idx.shape[0]
    V, D = emb_weight.shape
    b_per_w = B // NW

    mesh = plsc.VectorSubcoreMesh(core_axis_name="c", subcore_axis_name="s")

    @functools.partial(
        pl.kernel,
        mesh=mesh,
        out_type=jax.ShapeDtypeStruct((B, D), emb_weight.dtype),
        scratch_types=[
            pltpu.VMEM((b_per_w,), jnp.int32),
            pltpu.VMEM((b_per_w, D), emb_weight.dtype),
            pltpu.VMEM((L, D), emb_weight.dtype),
            pltpu.SemaphoreType.DMA,
        ],
        compiler_params=pltpu.CompilerParams(needs_layout_passes=False),
    )
    def gather_kernel(table_hbm, idx_hbm, out_hbm, idx_v, rows_v, drain_v,
                      sem):
        wid = lax.axis_index("s") * NC + lax.axis_index("c")
        base = wid * b_per_w
        pltpu.sync_copy(idx_hbm.at[pl.ds(base, b_per_w)], idx_v)
        lane = lax.iota(jnp.int32, L)

        @pl.loop(0, b_per_w, step=L)
        def _(i):
            v = idx_v[pl.ds(i, L)]
            for k in range(L):
                j = lax.reduce_sum(jnp.where(lane == k, v, 0), axes=(0,))
                pltpu.async_copy(
                    table_hbm.at[pl.ds(j, 1)],
                    rows_v.at[pl.ds(i + k, 1)],
                    sem,
                )

            @pl.when(i >= LAG * L)
            def _():
                pltpu.make_async_copy(
                    table_hbm.at[pl.ds(0, L)], drain_v, sem).wait()

        for _ in range(LAG):
            pltpu.make_async_copy(
                table_hbm.at[pl.ds(0, L)], drain_v, sem).wait()

        pltpu.sync_copy(rows_v, out_hbm.at[pl.ds(base, b_per_w)])

    return gather_kernel(emb_weight, idx.astype(jnp.int32))
